# baseline (speedup 1.0000x reference)
import functools

import jax
import jax.numpy as jnp
from jax.experimental import pallas as pl
from jax.experimental.pallas import tpu as pltpu


def _round_up(x: int, m: int) -> int:
    return (x + m - 1) // m * m


def _layer_norm(y, gamma, beta, eps):
    mean = jnp.mean(y, axis=-1, keepdims=True)
    c = y - mean
    var = jnp.mean(c * c, axis=-1, keepdims=True)
    return c * jax.lax.rsqrt(var + eps) * gamma + beta


def _conv_ln_dma(win_get, wbf_ref, xc_ref, sem, w_bf, b, gamma, beta,
                 rout, kw, npad, f, eps):
    copies = [
        pltpu.make_async_copy(
            wbf_ref.at[pl.ds(k * npad, rout)],
            xc_ref.at[pl.ds(0, rout), pl.ds(k * f, f)],
            sem.at[k])
        for k in range(kw)
    ]
    for c in copies:
        c.start()
    pool = win_get(0, rout)
    for k in range(1, kw):
        pool = jnp.maximum(pool, win_get(k * npad, rout))
    skip = win_get((kw - 1) * npad, rout)
    res = pool + skip
    for c in copies:
        c.wait()
    acc = jnp.dot(xc_ref[pl.ds(0, rout), :], w_bf,
                  preferred_element_type=jnp.float32)
    y = jnp.maximum(acc + b, 0.0) + res
    return _layer_norm(y, gamma, beta, eps)


def _fc_ln(x, w1, b1, w2, b2, gamma, beta, eps):
    h = jnp.dot(x.astype(jnp.bfloat16), w1, preferred_element_type=jnp.float32)
    h = jnp.maximum(h + b1, 0.0)
    y = jnp.dot(h.astype(jnp.bfloat16), w2, preferred_element_type=jnp.float32)
    y = y + b2 + x
    return _layer_norm(y, gamma, beta, eps)


def _encoder_kernel(x_ref, xh_ref, cw_ref, fw_ref, v_ref, o_ref,
                    wbf_ref, xc_ref, sem,
                    *, kw, npad, eps):
    r2 = o_ref.shape[0]
    f = o_ref.shape[-1]
    h = (kw - 1) * npad
    kf = kw * f
    r1 = r2 + h

    def v(i):
        return v_ref[8 * i:8 * i + 1]

    win0 = jnp.concatenate([x_ref[0], xh_ref[0]], axis=0)
    wbf_ref[...] = win0.astype(jnp.bfloat16)

    def win0_get(lo, n):
        return jax.lax.slice_in_dim(win0, lo, lo + n)

    z = _conv_ln_dma(win0_get, wbf_ref, xc_ref, sem, cw_ref[0:kf],
                     v(0), v(1), v(2), r1, kw, npad, f, eps)
    z = _fc_ln(z, fw_ref[0:f], v(3), fw_ref[f:2 * f], v(4), v(5), v(6), eps)

    wbf_ref[pl.ds(0, r1)] = z.astype(jnp.bfloat16)

    def win1_get(lo, n):
        return jax.lax.slice_in_dim(z, lo, lo + n)

    z = _conv_ln_dma(win1_get, wbf_ref, xc_ref, sem, cw_ref[kf:2 * kf],
                     v(7), v(8), v(9), r2, kw, npad, f, eps)
    z = _fc_ln(z, fw_ref[2 * f:3 * f], v(10), fw_ref[3 * f:4 * f], v(11),
               v(12), v(13), eps)
    o_ref[...] = z


def _fold_conv_w(cw, width, f):
    kw = cw.shape[0]
    eye = jnp.eye(width, dtype=jnp.float32)
    w = jnp.einsum("kio,wv->kiwov", cw.astype(jnp.float32), eye).reshape(kw * f, f)
    return w.astype(jnp.bfloat16)


def kernel(inputs, conv_w_0, conv_b_0, conv_gamma_0, conv_beta_0,
           fc_w1_0, fc_b1_0, fc_w2_0, fc_b2_0, fc_gamma_0, fc_beta_0,
           conv_w_1, conv_b_1, conv_gamma_1, conv_beta_1,
           fc_w1_1, fc_b1_1, fc_w2_1, fc_b2_1, fc_gamma_1, fc_beta_1):
    T, N, F = inputs.shape
    kw, ch, _ = conv_w_0.shape
    width = F // ch
    eps = 1e-5
    t_final = T - 2 * (kw - 1)

    npad = _round_up(N, 8)
    tb = 128
    num_t = -(-T // tb)
    tp = num_t * tb
    x = inputs
    if tp != T or npad != N:
        x = jnp.pad(x, ((0, tp - T), (0, npad - N), (0, 0)))
    rows_blk = tb * npad
    x3 = x.reshape(num_t, rows_blk, F)
    h2 = 2 * (kw - 1) * npad
    halo = jnp.concatenate([x3[1:, :h2], x3[-1:, :h2]], axis=0)

    cw = jnp.concatenate([_fold_conv_w(conv_w_0, width, F),
                          _fold_conv_w(conv_w_1, width, F)], axis=0)
    fw = jnp.concatenate([fc_w1_0, fc_w2_0, fc_w1_1, fc_w2_1],
                         axis=0).astype(jnp.bfloat16)
    rows = [jnp.repeat(conv_b_0, width), conv_gamma_0, conv_beta_0,
            fc_b1_0, fc_b2_0, fc_gamma_0, fc_beta_0,
            jnp.repeat(conv_b_1, width), conv_gamma_1, conv_beta_1,
            fc_b1_1, fc_b2_1, fc_gamma_1, fc_beta_1]
    vtab = jnp.zeros((8 * len(rows), F), jnp.float32)
    vtab = vtab.at[::8].set(jnp.stack([r.astype(jnp.float32) for r in rows]))

    def const_spec(shape):
        return pl.BlockSpec(shape, lambda j: tuple(0 for _ in shape))

    in_specs = [
        pl.BlockSpec((1, rows_blk, F), lambda j: (j, 0, 0)),
        pl.BlockSpec((1, h2, F), lambda j: (j, 0, 0)),
        const_spec((2 * kw * F, F)),
        const_spec((4 * F, F)),
        const_spec((112, F)),
    ]
    r1 = rows_blk + (kw - 1) * npad

    out2d = pl.pallas_call(
        functools.partial(_encoder_kernel, kw=kw, npad=npad, eps=eps),
        out_shape=jax.ShapeDtypeStruct((t_final * npad, F), inputs.dtype),
        grid_spec=pltpu.PrefetchScalarGridSpec(
            num_scalar_prefetch=0,
            grid=(num_t,),
            in_specs=in_specs,
            out_specs=pl.BlockSpec((rows_blk, F), lambda j: (j, 0)),
            scratch_shapes=[
                pltpu.VMEM((rows_blk + h2, F), jnp.bfloat16),
                pltpu.VMEM((r1, kw * F), jnp.bfloat16),
                pltpu.SemaphoreType.DMA((kw,)),
            ],
        ),
        compiler_params=pltpu.CompilerParams(
            dimension_semantics=("parallel",),
            vmem_limit_bytes=56 << 20),
        cost_estimate=pl.CostEstimate(
            flops=2 * tp * npad * F * F * (2 * kw + 4),
            transcendentals=4 * tp * npad,
            bytes_accessed=2 * tp * npad * F * 4 + (2 * kw + 4) * F * F * 2),
    )(x3, halo, cw, fw, vtab)

    return out2d.reshape(t_final, npad, F)[:, :N, :]


# speedup vs baseline: 1.1773x; 1.1773x over previous
import functools

import jax
import jax.numpy as jnp
from jax.experimental import pallas as pl
from jax.experimental.pallas import tpu as pltpu


def _round_up(x: int, m: int) -> int:
    return (x + m - 1) // m * m


def _layer_norm(y, gamma, beta, eps):
    mean = jnp.mean(y, axis=-1, keepdims=True)
    c = y - mean
    var = jnp.mean(c * c, axis=-1, keepdims=True)
    return c * jax.lax.rsqrt(var + eps) * gamma + beta


def _conv_ln(win, w_bf, b, gamma, beta, rout, kw, npad, eps):
    win_bf = win.astype(jnp.bfloat16)
    xc = jnp.concatenate([win_bf[k * npad:k * npad + rout] for k in range(kw)],
                         axis=1)
    acc = jnp.dot(xc, w_bf, preferred_element_type=jnp.float32)
    pool = win[0:rout]
    for k in range(1, kw):
        pool = jnp.maximum(pool, win[k * npad:k * npad + rout])
    y = jnp.maximum(acc + b, 0.0) + pool + win[(kw - 1) * npad:(kw - 1) * npad + rout]
    return _layer_norm(y, gamma, beta, eps)


def _fc_ln(x, w1, b1, w2, b2, gamma, beta, eps):
    h = jnp.dot(x.astype(jnp.bfloat16), w1, preferred_element_type=jnp.float32)
    h = jnp.maximum(h + b1, 0.0)
    y = jnp.dot(h.astype(jnp.bfloat16), w2, preferred_element_type=jnp.float32)
    y = y + b2 + x
    return _layer_norm(y, gamma, beta, eps)


def _encoder_kernel(x_ref, xh_ref, cw_ref, fw_ref, v_ref, o_ref,
                    *, kw, npad, eps):
    r2 = o_ref.shape[0]
    f = o_ref.shape[-1]
    h = (kw - 1) * npad
    kf = kw * f
    win0 = jnp.concatenate([x_ref[0], xh_ref[0]], axis=0)

    def v(i):
        return v_ref[8 * i:8 * i + 1]

    z = _conv_ln(win0, cw_ref[0:kf], v(0), v(1), v(2), r2 + h, kw, npad, eps)
    z = _fc_ln(z, fw_ref[0:f], v(3), fw_ref[f:2 * f], v(4), v(5), v(6), eps)
    z = _conv_ln(z, cw_ref[kf:2 * kf], v(7), v(8), v(9), r2, kw, npad, eps)
    z = _fc_ln(z, fw_ref[2 * f:3 * f], v(10), fw_ref[3 * f:4 * f], v(11),
               v(12), v(13), eps)
    o_ref[...] = z


def _fold_conv_w(cw, width, f):
    kw = cw.shape[0]
    eye = jnp.eye(width, dtype=jnp.float32)
    w = jnp.einsum("kio,wv->kiwov", cw.astype(jnp.float32), eye).reshape(kw * f, f)
    return w.astype(jnp.bfloat16)


def kernel(inputs, conv_w_0, conv_b_0, conv_gamma_0, conv_beta_0,
           fc_w1_0, fc_b1_0, fc_w2_0, fc_b2_0, fc_gamma_0, fc_beta_0,
           conv_w_1, conv_b_1, conv_gamma_1, conv_beta_1,
           fc_w1_1, fc_b1_1, fc_w2_1, fc_b2_1, fc_gamma_1, fc_beta_1):
    T, N, F = inputs.shape
    kw, ch, _ = conv_w_0.shape
    width = F // ch
    eps = 1e-5
    t_final = T - 2 * (kw - 1)

    npad = _round_up(N, 8)
    tb = 128
    num_t = -(-T // tb)
    tp = num_t * tb
    x = inputs
    if tp != T or npad != N:
        x = jnp.pad(x, ((0, tp - T), (0, npad - N), (0, 0)))
    rows_blk = tb * npad
    x3 = x.reshape(num_t, rows_blk, F)
    h2 = 2 * (kw - 1) * npad
    halo = jnp.concatenate([x3[1:, :h2], x3[-1:, :h2]], axis=0)

    cw = jnp.concatenate([_fold_conv_w(conv_w_0, width, F),
                          _fold_conv_w(conv_w_1, width, F)], axis=0)
    fw = jnp.concatenate([fc_w1_0, fc_w2_0, fc_w1_1, fc_w2_1],
                         axis=0).astype(jnp.bfloat16)
    rows = [jnp.repeat(conv_b_0, width), conv_gamma_0, conv_beta_0,
            fc_b1_0, fc_b2_0, fc_gamma_0, fc_beta_0,
            jnp.repeat(conv_b_1, width), conv_gamma_1, conv_beta_1,
            fc_b1_1, fc_b2_1, fc_gamma_1, fc_beta_1]
    vtab = jnp.zeros((8 * len(rows), F), jnp.float32)
    vtab = vtab.at[::8].set(jnp.stack([r.astype(jnp.float32) for r in rows]))

    def const_spec(shape):
        return pl.BlockSpec(shape, lambda j: tuple(0 for _ in shape))

    in_specs = [
        pl.BlockSpec((1, rows_blk, F), lambda j: (j, 0, 0)),
        pl.BlockSpec((1, h2, F), lambda j: (j, 0, 0)),
        const_spec((2 * kw * F, F)),
        const_spec((4 * F, F)),
        const_spec((112, F)),
    ]

    out2d = pl.pallas_call(
        functools.partial(_encoder_kernel, kw=kw, npad=npad, eps=eps),
        out_shape=jax.ShapeDtypeStruct((t_final * npad, F), inputs.dtype),
        grid_spec=pltpu.PrefetchScalarGridSpec(
            num_scalar_prefetch=0,
            grid=(num_t,),
            in_specs=in_specs,
            out_specs=pl.BlockSpec((rows_blk, F), lambda j: (j, 0)),
        ),
        compiler_params=pltpu.CompilerParams(
            dimension_semantics=("parallel",),
            vmem_limit_bytes=56 << 20),
        cost_estimate=pl.CostEstimate(
            flops=2 * tp * npad * F * F * (2 * kw + 4),
            transcendentals=4 * tp * npad,
            bytes_accessed=2 * tp * npad * F * 4 + (2 * kw + 4) * F * F * 2),
    )(x3, halo, cw, fw, vtab)

    return out2d.reshape(t_final, npad, F)[:, :N, :]
